# baseline (speedup 1.0000x reference)
import functools
import jax
import jax.numpy as jnp
from jax.experimental import pallas as pl
from jax.experimental.pallas import tpu as pltpu

_C = 256
_GROUPS = 32
_GSIZE = _C // _GROUPS
_EPS = 1e-5


def _group_mat():
    r = jax.lax.broadcasted_iota(jnp.int32, (_C, _C), 0) // _GSIZE
    c = jax.lax.broadcasted_iota(jnp.int32, (_C, _C), 1) // _GSIZE
    return (r == c).astype(jnp.float32)


def _chunks(H, W):
    ch = max(1, min(H, 512 // W))
    return [(h0, min(ch, H - h0)) for h0 in range(0, H, ch)]


def _conv_chunk(b_ref, w_ref, h0, ch, W, layer=None):
    a = None
    for k in range(9):
        win = b_ref[k % 3, (h0 + k // 3) * W:(h0 + k // 3 + ch) * W, :]
        wk = (w_ref[layer, k * _C:(k + 1) * _C, :] if layer is not None
              else w_ref[k * _C:(k + 1) * _C, :])
        t = jnp.dot(win, wk, preferred_element_type=jnp.float32)
        a = t if a is None else a + t
    return a


def _repack(b_ref, pad_ref, H, W):
    for kw in range(3):
        b_ref[kw] = pad_ref[0:H + 2, kw:kw + W, :].reshape((H + 2) * W, _C)


def _tower(feat_ref, tw_ref, lp_ref, hw_ref, hb_ref, out_ref, pad_ref,
           b_ref, acc_ref, bb_scale, H, W, head_co, bbox):
    N = H * W
    P = _group_mat()
    chunks = _chunks(H, W)

    pad_ref[...] = jnp.zeros_like(pad_ref)
    pad_ref[1:H + 1, 1:W + 1, :] = feat_ref[0]

    for layer in range(tw_ref.shape[0]):
        _repack(b_ref, pad_ref, H, W)
        s = q = None
        for h0, ch in chunks:
            a = _conv_chunk(b_ref, tw_ref, h0, ch, W, layer=layer)
            acc_ref[h0 * W:(h0 + ch) * W, :] = a
            cs = jnp.sum(a, axis=0, keepdims=True)
            cq = jnp.sum(a * a, axis=0, keepdims=True)
            s = cs if s is None else s + cs
            q = cq if q is None else q + cq
        lp = lp_ref[layer]
        b, gamma, beta = lp[0:1], lp[1:2], lp[2:3]
        gsb, gsb2 = lp[3:4], lp[4:5]
        stats = jnp.concatenate([s, q, b * s], axis=0)
        gs = jnp.dot(stats, P, preferred_element_type=jnp.float32)
        inv_n = 1.0 / (_GSIZE * N)
        mu = (gs[0:1] + N * gsb) * inv_n
        ey2 = (gs[1:2] + 2.0 * gs[2:3] + N * gsb2) * inv_n
        rstd = jax.lax.rsqrt(ey2 - mu * mu + _EPS)
        sc = rstd * gamma
        sh = (b - mu) * sc + beta
        for h0, ch in chunks:
            a = acc_ref[h0 * W:(h0 + ch) * W, :]
            x = jnp.maximum(a * sc + sh, 0.0).astype(jnp.bfloat16)
            pad_ref[h0 + 1:h0 + ch + 1, 1:W + 1, :] = x.reshape(ch, W, _C)

    _repack(b_ref, pad_ref, H, W)
    for h0, ch in chunks:
        y = _conv_chunk(b_ref, hw_ref, h0, ch, W) + hb_ref[0:1]
        if bbox:
            y = jnp.exp(y * bb_scale)
        out_ref[0, h0:h0 + ch] = y.reshape(ch, W, head_co)


def _mega_kernel(*refs, dims):
    nl = len(dims)
    feats = refs[0:nl]
    ctw, clp, chw, chb, btw, blp, bhw, bhb, scs = refs[nl:nl + 9]
    outs = refs[nl + 9:nl + 9 + 2 * nl]
    scr = refs[nl + 9 + 2 * nl:]
    pads, bufs, accs = scr[0:nl], scr[nl:2 * nl], scr[2 * nl:3 * nl]
    for l, (H, W) in enumerate(dims):
        _tower(feats[l], ctw, clp, chw, chb, outs[2 * l],
               pads[l], bufs[l], accs[l], None, H, W, 81, False)
        _tower(feats[l], btw, blp, bhw, bhb, outs[2 * l + 1],
               pads[l], bufs[l], accs[l], scs[l:l + 1, 0:1], H, W, 4,
               True)


def _run_all(feats, cls_p, box_p, scales):
    B = feats[0].shape[0]
    dims = [(f.shape[1], f.shape[2]) for f in feats]
    kern = functools.partial(_mega_kernel, dims=dims)
    full = lambda a: pl.BlockSpec(a.shape, lambda b: (0,) * a.ndim)
    wargs = list(cls_p) + list(box_p) + [scales]
    in_specs = ([pl.BlockSpec((1, H, W, _C), lambda b: (b, 0, 0, 0))
                 for (H, W) in dims] + [full(a) for a in wargs])
    out_specs, out_shape, scratch = [], [], []
    for (H, W) in dims:
        for co in (81, 4):
            out_specs.append(
                pl.BlockSpec((1, H, W, co), lambda b: (b, 0, 0, 0)))
            out_shape.append(
                jax.ShapeDtypeStruct((B, H, W, co), jnp.float32))
    for (H, W) in dims:
        scratch.append(pltpu.VMEM((H + 2, W + 2, _C), jnp.bfloat16))
    for (H, W) in dims:
        scratch.append(pltpu.VMEM((3, (H + 2) * W, _C), jnp.bfloat16))
    for (H, W) in dims:
        scratch.append(pltpu.VMEM((H * W, _C), jnp.float32))
    return pl.pallas_call(
        kern,
        grid=(B,),
        in_specs=in_specs,
        out_specs=out_specs,
        out_shape=out_shape,
        scratch_shapes=scratch,
        compiler_params=pltpu.CompilerParams(
            dimension_semantics=("parallel",)),
    )(*feats, *wargs)


def _gs_vec(v):
    return jnp.repeat(v.reshape(_GROUPS, _GSIZE).sum(axis=1), _GSIZE)


def _prep_tower(layers):
    ws, lps = [], []
    for l in layers:
        ws.append(jnp.transpose(l['w'], (2, 3, 1, 0)).reshape(9 * _C, _C))
        b, g, beta = l['b'], l['g'], l['beta']
        lps.append(jnp.stack([b, g, beta, _gs_vec(b), _gs_vec(b * b),
                              jnp.zeros_like(b), jnp.zeros_like(b),
                              jnp.zeros_like(b)]))
    return (jnp.stack(ws).astype(jnp.bfloat16),
            jnp.stack(lps).astype(jnp.float32))


def _prep_head(w):
    co = w.shape[0]
    return jnp.transpose(w, (2, 3, 1, 0)).reshape(9 * _C, co).astype(
        jnp.bfloat16)


def kernel(features, params):
    cls_tw, cls_lp = _prep_tower(params['cls_tower'])
    box_tw, box_lp = _prep_tower(params['bbox_tower'])
    cls_hw = _prep_head(jnp.concatenate(
        [params['cls_logits']['w'], params['centerness']['w']], axis=0))
    cls_hb = jnp.concatenate(
        [params['cls_logits']['b'], params['centerness']['b']])[None, :]
    box_hw = _prep_head(params['bbox_pred']['w'])
    box_hb = params['bbox_pred']['b'][None, :]
    cls_p = (cls_tw, cls_lp, cls_hw, cls_hb)
    box_p = (box_tw, box_lp, box_hw, box_hb)
    scales = jnp.stack(
        [params['scales'][l].reshape(1) for l in range(len(features))])

    feats = [jnp.transpose(f, (0, 2, 3, 1)).astype(jnp.bfloat16)
             for f in features]
    ys = _run_all(feats, cls_p, box_p, scales.astype(jnp.float32))

    logits, bbox, ctr = [], [], []
    for l in range(len(features)):
        yc, yb = ys[2 * l], ys[2 * l + 1]
        logits.append(jnp.transpose(yc[..., :80], (0, 3, 1, 2)))
        ctr.append(jnp.transpose(yc[..., 80:81], (0, 3, 1, 2)))
        bbox.append(jnp.transpose(yb, (0, 3, 1, 2)))
    return tuple(logits), tuple(bbox), tuple(ctr)


# speedup vs baseline: 1.0245x; 1.0245x over previous
import functools
import jax
import jax.numpy as jnp
from jax.experimental import pallas as pl
from jax.experimental.pallas import tpu as pltpu

_C = 256
_GROUPS = 32
_GSIZE = _C // _GROUPS
_EPS = 1e-5


def _group_mat():
    r = jax.lax.broadcasted_iota(jnp.int32, (_C, _C), 0) // _GSIZE
    c = jax.lax.broadcasted_iota(jnp.int32, (_C, _C), 1) // _GSIZE
    return (r == c).astype(jnp.float32)


def _chunks(H, W):
    ch = max(1, min(H, 128 // W))
    return [(h0, min(ch, H - h0)) for h0 in range(0, H, ch)]


def _conv_chunk(b_ref, w_ref, h0, ch, W, layer=None):
    a = None
    for k in range(9):
        win = b_ref[k % 3, (h0 + k // 3) * W:(h0 + k // 3 + ch) * W, :]
        wk = (w_ref[layer, k * _C:(k + 1) * _C, :] if layer is not None
              else w_ref[k * _C:(k + 1) * _C, :])
        t = jnp.dot(win, wk, preferred_element_type=jnp.float32)
        a = t if a is None else a + t
    return a


def _repack(b_ref, pad_ref, H, W):
    for kw in range(3):
        b_ref[kw] = pad_ref[0:H + 2, kw:kw + W, :].reshape((H + 2) * W, _C)


def _tower(feat_ref, tw_ref, lp_ref, hw_ref, hb_ref, out_ref, pad_ref,
           b_ref, acc_ref, bb_scale, H, W, head_co, bbox):
    N = H * W
    P = _group_mat()
    chunks = _chunks(H, W)

    pad_ref[...] = jnp.zeros_like(pad_ref)
    pad_ref[1:H + 1, 1:W + 1, :] = feat_ref[0]

    for layer in range(tw_ref.shape[0]):
        _repack(b_ref, pad_ref, H, W)
        s = q = None
        for h0, ch in chunks:
            a = _conv_chunk(b_ref, tw_ref, h0, ch, W, layer=layer)
            acc_ref[h0 * W:(h0 + ch) * W, :] = a
            cs = jnp.sum(a, axis=0, keepdims=True)
            cq = jnp.sum(a * a, axis=0, keepdims=True)
            s = cs if s is None else s + cs
            q = cq if q is None else q + cq
        lp = lp_ref[layer]
        b, gamma, beta = lp[0:1], lp[1:2], lp[2:3]
        gsb, gsb2 = lp[3:4], lp[4:5]
        stats = jnp.concatenate([s, q, b * s], axis=0)
        gs = jnp.dot(stats, P, preferred_element_type=jnp.float32)
        inv_n = 1.0 / (_GSIZE * N)
        mu = (gs[0:1] + N * gsb) * inv_n
        ey2 = (gs[1:2] + 2.0 * gs[2:3] + N * gsb2) * inv_n
        rstd = jax.lax.rsqrt(ey2 - mu * mu + _EPS)
        sc = rstd * gamma
        sh = (b - mu) * sc + beta
        for h0, ch in chunks:
            a = acc_ref[h0 * W:(h0 + ch) * W, :]
            x = jnp.maximum(a * sc + sh, 0.0).astype(jnp.bfloat16)
            pad_ref[h0 + 1:h0 + ch + 1, 1:W + 1, :] = x.reshape(ch, W, _C)

    _repack(b_ref, pad_ref, H, W)
    for h0, ch in chunks:
        y = _conv_chunk(b_ref, hw_ref, h0, ch, W) + hb_ref[0:1]
        if bbox:
            y = jnp.exp(y * bb_scale)
        out_ref[0, h0:h0 + ch] = y.reshape(ch, W, head_co)


def _mega_kernel(*refs, dims):
    nl = len(dims)
    feats = refs[0:nl]
    ctw, clp, chw, chb, btw, blp, bhw, bhb, scs = refs[nl:nl + 9]
    outs = refs[nl + 9:nl + 9 + 2 * nl]
    scr = refs[nl + 9 + 2 * nl:]
    pads, bufs, accs = scr[0:nl], scr[nl:2 * nl], scr[2 * nl:3 * nl]
    for l, (H, W) in enumerate(dims):
        _tower(feats[l], ctw, clp, chw, chb, outs[2 * l],
               pads[l], bufs[l], accs[l], None, H, W, 81, False)
        _tower(feats[l], btw, blp, bhw, bhb, outs[2 * l + 1],
               pads[l], bufs[l], accs[l], scs[l:l + 1, 0:1], H, W, 4,
               True)


def _run_all(feats, cls_p, box_p, scales):
    B = feats[0].shape[0]
    dims = [(f.shape[1], f.shape[2]) for f in feats]
    kern = functools.partial(_mega_kernel, dims=dims)
    full = lambda a: pl.BlockSpec(a.shape, lambda b: (0,) * a.ndim)
    wargs = list(cls_p) + list(box_p) + [scales]
    in_specs = ([pl.BlockSpec((1, H, W, _C), lambda b: (b, 0, 0, 0))
                 for (H, W) in dims] + [full(a) for a in wargs])
    out_specs, out_shape, scratch = [], [], []
    for (H, W) in dims:
        for co in (81, 4):
            out_specs.append(
                pl.BlockSpec((1, H, W, co), lambda b: (b, 0, 0, 0)))
            out_shape.append(
                jax.ShapeDtypeStruct((B, H, W, co), jnp.float32))
    for (H, W) in dims:
        scratch.append(pltpu.VMEM((H + 2, W + 2, _C), jnp.bfloat16))
    for (H, W) in dims:
        scratch.append(pltpu.VMEM((3, (H + 2) * W, _C), jnp.bfloat16))
    for (H, W) in dims:
        scratch.append(pltpu.VMEM((H * W, _C), jnp.float32))
    return pl.pallas_call(
        kern,
        grid=(B,),
        in_specs=in_specs,
        out_specs=out_specs,
        out_shape=out_shape,
        scratch_shapes=scratch,
        compiler_params=pltpu.CompilerParams(
            dimension_semantics=("parallel",)),
    )(*feats, *wargs)


def _gs_vec(v):
    return jnp.repeat(v.reshape(_GROUPS, _GSIZE).sum(axis=1), _GSIZE)


def _prep_tower(layers):
    ws, lps = [], []
    for l in layers:
        ws.append(jnp.transpose(l['w'], (2, 3, 1, 0)).reshape(9 * _C, _C))
        b, g, beta = l['b'], l['g'], l['beta']
        lps.append(jnp.stack([b, g, beta, _gs_vec(b), _gs_vec(b * b),
                              jnp.zeros_like(b), jnp.zeros_like(b),
                              jnp.zeros_like(b)]))
    return (jnp.stack(ws).astype(jnp.bfloat16),
            jnp.stack(lps).astype(jnp.float32))


def _prep_head(w):
    co = w.shape[0]
    return jnp.transpose(w, (2, 3, 1, 0)).reshape(9 * _C, co).astype(
        jnp.bfloat16)


def kernel(features, params):
    cls_tw, cls_lp = _prep_tower(params['cls_tower'])
    box_tw, box_lp = _prep_tower(params['bbox_tower'])
    cls_hw = _prep_head(jnp.concatenate(
        [params['cls_logits']['w'], params['centerness']['w']], axis=0))
    cls_hb = jnp.concatenate(
        [params['cls_logits']['b'], params['centerness']['b']])[None, :]
    box_hw = _prep_head(params['bbox_pred']['w'])
    box_hb = params['bbox_pred']['b'][None, :]
    cls_p = (cls_tw, cls_lp, cls_hw, cls_hb)
    box_p = (box_tw, box_lp, box_hw, box_hb)
    scales = jnp.stack(
        [params['scales'][l].reshape(1) for l in range(len(features))])

    feats = [jnp.transpose(f, (0, 2, 3, 1)).astype(jnp.bfloat16)
             for f in features]
    ys = _run_all(feats, cls_p, box_p, scales.astype(jnp.float32))

    logits, bbox, ctr = [], [], []
    for l in range(len(features)):
        yc, yb = ys[2 * l], ys[2 * l + 1]
        logits.append(jnp.transpose(yc[..., :80], (0, 3, 1, 2)))
        ctr.append(jnp.transpose(yc[..., 80:81], (0, 3, 1, 2)))
        bbox.append(jnp.transpose(yb, (0, 3, 1, 2)))
    return tuple(logits), tuple(bbox), tuple(ctr)
